# baseline (speedup 1.0000x reference)
import functools

import jax
import jax.numpy as jnp
from jax import lax
from jax.experimental import pallas as pl
from jax.experimental.pallas import tpu as pltpu
from jax.experimental.pallas import tpu_sc as plsc

_N = 10000
_E = 320000
_TILE = 400
_GRID = _N // _TILE

_f32 = jnp.float32


def _dot_t(a, w):
    return lax.dot_general(a, w, (((1,), (1,)), ((), ())),
                           preferred_element_type=_f32)


def _rc_from_cnt(cnt_ref):
    cnt = cnt_ref[...][:, 0:1]
    return 1.0 / jnp.maximum(cnt, 1.0)


def _tc1_body(acc_ref, cnt_ref, x_ref, ewl_ref, ewr_ref, eb_ref,
              cwl_ref, cwr_ref, cb_ref, he_ref, hc_ref):
    rc = _rc_from_cnt(cnt_ref)
    agg = acc_ref[...] * rc
    x = x_ref[...]
    he = jnp.maximum(_dot_t(agg, ewl_ref[...]) + _dot_t(x, ewr_ref[...])
                     + eb_ref[...][None, :], 0.0)
    hc = jnp.maximum(_dot_t(agg, cwl_ref[...]) + _dot_t(x, cwr_ref[...])
                     + cb_ref[...][None, :], 0.0)
    he_ref[0] = he[:, :128]
    he_ref[1] = he[:, 128:]
    hc_ref[0] = hc[:, :128]
    hc_ref[1] = hc[:, 128:]


def _tc1(acc, cnt, x, ewl, ewr, eb, cwl, cwr, cb):
    T = _TILE
    row3 = pl.BlockSpec((2, T, 128), lambda i: (0, i, 0))
    return pl.pallas_call(
        _tc1_body,
        grid=(_GRID,),
        in_specs=[
            pl.BlockSpec((T, 128), lambda i: (i, 0)),
            pl.BlockSpec((T, 128), lambda i: (i, 0)),
            pl.BlockSpec((T, 128), lambda i: (i, 0)),
            pl.BlockSpec((256, 128), lambda i: (0, 0)),
            pl.BlockSpec((256, 128), lambda i: (0, 0)),
            pl.BlockSpec((256,), lambda i: (0,)),
            pl.BlockSpec((256, 128), lambda i: (0, 0)),
            pl.BlockSpec((256, 128), lambda i: (0, 0)),
            pl.BlockSpec((256,), lambda i: (0,)),
        ],
        out_specs=[row3, row3],
        out_shape=[jax.ShapeDtypeStruct((2, _N, 128), _f32),
                   jax.ShapeDtypeStruct((2, _N, 128), _f32)],
    )(acc, cnt, x, ewl, ewr, eb, cwl, cwr, cb)


def _tc2_body(acc_ref, h_ref, cnt_ref, noise_ref, wl_ref, wr_ref, b_ref,
              regw_ref, regb_ref, g1w_ref, g1b_ref, deg_ref, g1_ref):
    rc = _rc_from_cnt(cnt_ref)
    wl = wl_ref[...]
    wr = wr_ref[...]
    z = (_dot_t(acc_ref[0] * rc, wl[:, :128]) +
         _dot_t(acc_ref[1] * rc, wl[:, 128:]) +
         _dot_t(h_ref[0], wr[:, :128]) +
         _dot_t(h_ref[1], wr[:, 128:]) + b_ref[...][None, :])
    d = jnp.sum(z * regw_ref[...], axis=1, keepdims=True) + regb_ref[0]
    deg_ref[...] = jnp.maximum(d, 0.0)
    g1_ref[...] = jnp.maximum(_dot_t(z + noise_ref[...], g1w_ref[...])
                              + g1b_ref[...][None, :], 0.0)


def _tc2(acc, h, cnt, noise, wl, wr, b, regw, regb, g1w, g1b):
    T = _TILE
    row3 = pl.BlockSpec((2, T, 128), lambda i: (0, i, 0))
    return pl.pallas_call(
        _tc2_body,
        grid=(_GRID,),
        in_specs=[
            row3, row3,
            pl.BlockSpec((T, 128), lambda i: (i, 0)),
            pl.BlockSpec((T, 128), lambda i: (i, 0)),
            pl.BlockSpec((128, 256), lambda i: (0, 0)),
            pl.BlockSpec((128, 256), lambda i: (0, 0)),
            pl.BlockSpec((128,), lambda i: (0,)),
            pl.BlockSpec((1, 128), lambda i: (0, 0)),
            pl.BlockSpec((1,), lambda i: (0,)),
            pl.BlockSpec((256, 128), lambda i: (0, 0)),
            pl.BlockSpec((256,), lambda i: (0,)),
        ],
        out_specs=[pl.BlockSpec((T, 1), lambda i: (i, 0)),
                   pl.BlockSpec((T, 256), lambda i: (i, 0))],
        out_shape=[jax.ShapeDtypeStruct((_N, 1), _f32),
                   jax.ShapeDtypeStruct((_N, 256), _f32)],
    )(acc, h, cnt, noise, wl, wr, b, regw, regb, g1w, g1b)


def _tc3_body(g1_ref, g2w_ref, g2b_ref, gfw_ref, gfb_ref, out_ref):
    g2 = jnp.maximum(_dot_t(g1_ref[...], g2w_ref[...]) + g2b_ref[...][None, :],
                     0.0)
    out_ref[...] = jnp.tanh(_dot_t(g2, gfw_ref[...]) + gfb_ref[...][None, :])


def _tc3(g1, g2w, g2b, gfw, gfb):
    T = _TILE
    return pl.pallas_call(
        _tc3_body,
        grid=(_GRID,),
        in_specs=[
            pl.BlockSpec((T, 256), lambda i: (i, 0)),
            pl.BlockSpec((2048, 256), lambda i: (0, 0)),
            pl.BlockSpec((2048,), lambda i: (0,)),
            pl.BlockSpec((640, 2048), lambda i: (0, 0)),
            pl.BlockSpec((640,), lambda i: (0,)),
        ],
        out_specs=pl.BlockSpec((T, 640), lambda i: (i, 0)),
        out_shape=jax.ShapeDtypeStruct((_N, 640), _f32),
    )(g1, g2w, g2b, gfw, gfb)


def _tc4_body(acc_ref, h_ref, cnt_ref, wl_ref, wr_ref, b_ref, nc_ref):
    rc = _rc_from_cnt(cnt_ref)
    wl = wl_ref[...]
    wr = wr_ref[...]
    nc_ref[...] = (_dot_t(acc_ref[0] * rc, wl[:, :128]) +
                   _dot_t(acc_ref[1] * rc, wl[:, 128:]) +
                   _dot_t(h_ref[0], wr[:, :128]) +
                   _dot_t(h_ref[1], wr[:, 128:]) + b_ref[...][None, :])


def _tc4(acc, h, cnt, wl, wr, b):
    T = _TILE
    row3 = pl.BlockSpec((2, T, 128), lambda i: (0, i, 0))
    return pl.pallas_call(
        _tc4_body,
        grid=(_GRID,),
        in_specs=[
            row3, row3,
            pl.BlockSpec((T, 128), lambda i: (i, 0)),
            pl.BlockSpec((128, 256), lambda i: (0, 0)),
            pl.BlockSpec((128, 256), lambda i: (0, 0)),
            pl.BlockSpec((128,), lambda i: (0,)),
        ],
        out_specs=pl.BlockSpec((T, 128), lambda i: (i, 0)),
        out_shape=jax.ShapeDtypeStruct((_N, 128), _f32),
    )(acc, h, cnt, wl, wr, b)


_B = 128
_EP = 327680
_HP = 5120
_AR = 5248
_RIN = _AR // 16
_RWB = _HP // 16
_K = 4
_NSB = _EP // _B // _K // 16


@functools.lru_cache(maxsize=None)
def _sc_mesh():
    return plsc.VectorSubcoreMesh(core_axis_name="c", subcore_axis_name="s")


def _remap_dst(dstv, part_base):
    basev = jnp.full((16,), part_base, jnp.int32)
    dumpv = jnp.full((16,), _HP, jnp.int32)
    for j in range(_B // 16):
        sl = pl.ds(j * 16, 16)
        local = dstv[sl] - basev
        ok = (local >= 0) & (local < _HP)
        dstv[sl] = jnp.where(ok, local, dumpv)


def _agg_pass(c, s, tab_h, src_h, dst_h, srcv, dstv, rows, accs, semg, sems,
              part_base, src_off):
    def blk(i, carry):
        m = s + 16 * i
        pltpu.sync_copy(src_h.at[pl.ds(m * _K, _K)], srcv)
        if src_off is not None:
            for j in range(_K):
                for q in range(_B // 16):
                    sl = pl.ds(q * 16, 16)
                    srcv[j, sl] = srcv[j, sl] + src_off
        gd = [pltpu.async_copy(tab_h.at[srcv.at[j]], rows.at[j], semg)
              for j in range(_K)]
        pltpu.sync_copy(dst_h.at[pl.ds(m * _K, _K)], dstv)
        for j in range(_K):
            _remap_dst(dstv.at[j], part_base)
        for d in gd:
            d.wait()
        sd = [pltpu.async_copy(rows.at[j], accs.at[dstv.at[j]], sems, add=True)
              for j in range(_K)]
        for d in sd:
            d.wait()
        return carry

    lax.fori_loop(0, _NSB, blk, 0)


def _slab_init(s, z_h, dst):
    r0 = s * _RIN
    pltpu.sync_copy(z_h.at[pl.ds(r0, _RIN)], dst.at[pl.ds(r0, _RIN)])


def _slab_wb(s, src, out_h, row_base, lead=None):
    r0 = s * _RWB
    if lead is None:
        dst = out_h.at[pl.ds(row_base + r0, _RWB)]
    else:
        dst = out_h.at[lead, pl.ds(row_base + r0, _RWB)]
    pltpu.sync_copy(src.at[pl.ds(r0, _RWB)], dst)


def _sc0_body(ones_h, dst_h, zacc_h, cnt_o, dstv, onesv, accs, sems):
    c = lax.axis_index("c")
    s = lax.axis_index("s")
    _slab_init(s, zacc_h, accs)
    pltpu.sync_copy(ones_h, onesv)
    plsc.subcore_barrier()

    def blk(i, carry):
        m = s + 16 * i
        pltpu.sync_copy(dst_h.at[pl.ds(m * _K, _K)], dstv)
        for j in range(_K):
            _remap_dst(dstv.at[j], c * _HP)
        sd = [pltpu.async_copy(onesv, accs.at[dstv.at[j]], sems, add=True)
              for j in range(_K)]
        for d in sd:
            d.wait()
        return carry

    lax.fori_loop(0, _NSB, blk, 0)
    plsc.subcore_barrier()
    _slab_wb(s, accs, cnt_o, c * _HP)


def _sc0(ones128, dst, zacc):
    f = functools.partial(
        pl.kernel,
        out_type=jax.ShapeDtypeStruct((2 * _HP, 128), _f32),
        mesh=_sc_mesh(),
        scratch_types=[
            pltpu.VMEM((_K, _B), jnp.int32),
            pltpu.VMEM((_B, 128), _f32),
            pltpu.VMEM_SHARED((_AR, 128), _f32),
            pltpu.SemaphoreType.DMA,
        ],
    )(_sc0_body)
    return f(ones128, dst, zacc)


def _sc1_body(x_h, src_h, dst_h, zacc_h, acc_o, srcv, dstv, rows, accs,
              semg, sems):
    c = lax.axis_index("c")
    s = lax.axis_index("s")
    _slab_init(s, zacc_h, accs)
    plsc.subcore_barrier()
    _agg_pass(c, s, x_h, src_h, dst_h, srcv, dstv, rows, accs, semg, sems,
              part_base=c * _HP, src_off=None)
    plsc.subcore_barrier()
    _slab_wb(s, accs, acc_o, c * _HP)


def _sc1(x, src, dst, zacc):
    f = functools.partial(
        pl.kernel,
        out_type=jax.ShapeDtypeStruct((2 * _HP, 128), _f32),
        mesh=_sc_mesh(),
        scratch_types=[
            pltpu.VMEM((_K, _B), jnp.int32),
            pltpu.VMEM((_K, _B), jnp.int32),
            pltpu.VMEM((_K, _B, 128), _f32),
            pltpu.VMEM_SHARED((_AR, 128), _f32),
            pltpu.SemaphoreType.DMA,
            pltpu.SemaphoreType.DMA,
        ],
    )(_sc1_body)
    return f(x, src, dst, zacc)


def _sc2_body(tab_h, src_h, dst_h, zacc_h, acc_o,
              srcv, dstv, rows, accs, semg, sems):
    c = lax.axis_index("c")
    s = lax.axis_index("s")
    src_off = jnp.full((16,), c * _N, jnp.int32)
    for p in (0, 1):
        _slab_init(s, zacc_h, accs)
        plsc.subcore_barrier()
        _agg_pass(c, s, tab_h, src_h, dst_h, srcv, dstv, rows, accs,
                  semg, sems, part_base=p * _HP, src_off=src_off)
        plsc.subcore_barrier()
        _slab_wb(s, accs, acc_o, p * _HP, lead=c)
        plsc.subcore_barrier()


def _sc2(tab, src, dst, zacc):
    f = functools.partial(
        pl.kernel,
        out_type=jax.ShapeDtypeStruct((2, 2 * _HP, 128), _f32),
        mesh=_sc_mesh(),
        scratch_types=[
            pltpu.VMEM((_K, _B), jnp.int32),
            pltpu.VMEM((_K, _B), jnp.int32),
            pltpu.VMEM((_K, _B, 128), _f32),
            pltpu.VMEM_SHARED((_AR, 128), _f32),
            pltpu.SemaphoreType.DMA,
            pltpu.SemaphoreType.DMA,
        ],
    )(_sc2_body)
    return f(tab, src, dst, zacc)


def kernel(x, edge_index, enc_Wl1, enc_Wr1, enc_b1, enc_Wl2, enc_Wr2, enc_b2,
           reg_W, reg_b, g1_W, g1_b, g2_W, g2_b, gf_W, gf_b,
           cls_Wl1, cls_Wr1, cls_b1, cls_Wl2, cls_Wr2, cls_b2, noise):
    src = edge_index[0].astype(jnp.int32)
    dst = edge_index[1].astype(jnp.int32)
    pad = _EP - _E
    src = jnp.concatenate([src, jnp.zeros((pad,), jnp.int32)])
    dst = jnp.concatenate([dst, jnp.full((pad,), _N, jnp.int32)])
    src = src.reshape(_EP // _B, _B)
    dst = dst.reshape(_EP // _B, _B)
    zacc = jnp.zeros((_AR, 128), _f32)
    ones128 = jnp.ones((_B, 128), _f32)

    cnt = _sc0(ones128, dst, zacc)
    acc1 = _sc1(x, src, dst, zacc)
    h_enc, h_cls = _tc1(acc1, cnt, x, enc_Wl1, enc_Wr1, enc_b1,
                        cls_Wl1, cls_Wr1, cls_b1)
    acc2e = _sc2(h_enc.reshape(2 * _N, 128), src, dst, zacc)
    acc2c = _sc2(h_cls.reshape(2 * _N, 128), src, dst, zacc)
    degree, g1 = _tc2(acc2e, h_enc, cnt, noise, enc_Wl2, enc_Wr2, enc_b2,
                      reg_W, reg_b, g1_W, g1_b)
    nc = _tc4(acc2c, h_cls, cnt, cls_Wl2, cls_Wr2, cls_b2)
    gen_feat = _tc3(g1, g2_W, g2_b, gf_W, gf_b)
    return degree, gen_feat, nc


# speedup vs baseline: 1.0482x; 1.0482x over previous
import functools

import jax
import jax.numpy as jnp
from jax import lax
from jax.experimental import pallas as pl
from jax.experimental.pallas import tpu as pltpu
from jax.experimental.pallas import tpu_sc as plsc

_N = 10000
_E = 320000
_TILE = 400
_GRID = _N // _TILE

_f32 = jnp.float32


def _dot_t(a, w):
    return lax.dot_general(a, w, (((1,), (1,)), ((), ())),
                           preferred_element_type=_f32)


def _rc_from_cnt(cnt_ref):
    cnt = cnt_ref[...][:, 0:1]
    return 1.0 / jnp.maximum(cnt, 1.0)


def _tc1_body(acc_ref, cnt_ref, x_ref, ewl_ref, ewr_ref, eb_ref,
              cwl_ref, cwr_ref, cb_ref, he_ref, hc_ref):
    rc = _rc_from_cnt(cnt_ref)
    agg = acc_ref[...] * rc
    x = x_ref[...]
    he = jnp.maximum(_dot_t(agg, ewl_ref[...]) + _dot_t(x, ewr_ref[...])
                     + eb_ref[...][None, :], 0.0)
    hc = jnp.maximum(_dot_t(agg, cwl_ref[...]) + _dot_t(x, cwr_ref[...])
                     + cb_ref[...][None, :], 0.0)
    he_ref[0] = he[:, :128]
    he_ref[1] = he[:, 128:]
    hc_ref[0] = hc[:, :128]
    hc_ref[1] = hc[:, 128:]


def _tc1(acc, cnt, x, ewl, ewr, eb, cwl, cwr, cb):
    T = _TILE
    row3 = pl.BlockSpec((2, T, 128), lambda i: (0, i, 0))
    return pl.pallas_call(
        _tc1_body,
        grid=(_GRID,),
        in_specs=[
            pl.BlockSpec((T, 128), lambda i: (i, 0)),
            pl.BlockSpec((T, 128), lambda i: (i, 0)),
            pl.BlockSpec((T, 128), lambda i: (i, 0)),
            pl.BlockSpec((256, 128), lambda i: (0, 0)),
            pl.BlockSpec((256, 128), lambda i: (0, 0)),
            pl.BlockSpec((256,), lambda i: (0,)),
            pl.BlockSpec((256, 128), lambda i: (0, 0)),
            pl.BlockSpec((256, 128), lambda i: (0, 0)),
            pl.BlockSpec((256,), lambda i: (0,)),
        ],
        out_specs=[row3, row3],
        out_shape=[jax.ShapeDtypeStruct((2, _N, 128), _f32),
                   jax.ShapeDtypeStruct((2, _N, 128), _f32)],
    )(acc, cnt, x, ewl, ewr, eb, cwl, cwr, cb)


def _tc2_body(acc_ref, h_ref, cnt_ref, noise_ref, wl_ref, wr_ref, b_ref,
              regw_ref, regb_ref, g1w_ref, g1b_ref, deg_ref, g1_ref):
    rc = _rc_from_cnt(cnt_ref)
    wl = wl_ref[...]
    wr = wr_ref[...]
    z = (_dot_t(acc_ref[0] * rc, wl[:, :128]) +
         _dot_t(acc_ref[1] * rc, wl[:, 128:]) +
         _dot_t(h_ref[0], wr[:, :128]) +
         _dot_t(h_ref[1], wr[:, 128:]) + b_ref[...][None, :])
    d = jnp.sum(z * regw_ref[...], axis=1, keepdims=True) + regb_ref[0]
    deg_ref[...] = jnp.maximum(d, 0.0)
    g1_ref[...] = jnp.maximum(_dot_t(z + noise_ref[...], g1w_ref[...])
                              + g1b_ref[...][None, :], 0.0)


def _tc2(acc, h, cnt, noise, wl, wr, b, regw, regb, g1w, g1b):
    T = _TILE
    row3 = pl.BlockSpec((2, T, 128), lambda i: (0, i, 0))
    return pl.pallas_call(
        _tc2_body,
        grid=(_GRID,),
        in_specs=[
            row3, row3,
            pl.BlockSpec((T, 128), lambda i: (i, 0)),
            pl.BlockSpec((T, 128), lambda i: (i, 0)),
            pl.BlockSpec((128, 256), lambda i: (0, 0)),
            pl.BlockSpec((128, 256), lambda i: (0, 0)),
            pl.BlockSpec((128,), lambda i: (0,)),
            pl.BlockSpec((1, 128), lambda i: (0, 0)),
            pl.BlockSpec((1,), lambda i: (0,)),
            pl.BlockSpec((256, 128), lambda i: (0, 0)),
            pl.BlockSpec((256,), lambda i: (0,)),
        ],
        out_specs=[pl.BlockSpec((T, 1), lambda i: (i, 0)),
                   pl.BlockSpec((T, 256), lambda i: (i, 0))],
        out_shape=[jax.ShapeDtypeStruct((_N, 1), _f32),
                   jax.ShapeDtypeStruct((_N, 256), _f32)],
    )(acc, h, cnt, noise, wl, wr, b, regw, regb, g1w, g1b)


def _tc3_body(g1_ref, g2w_ref, g2b_ref, gfw_ref, gfb_ref, out_ref):
    g2 = jnp.maximum(_dot_t(g1_ref[...], g2w_ref[...]) + g2b_ref[...][None, :],
                     0.0)
    out_ref[...] = jnp.tanh(_dot_t(g2, gfw_ref[...]) + gfb_ref[...][None, :])


def _tc3(g1, g2w, g2b, gfw, gfb):
    T = _TILE
    return pl.pallas_call(
        _tc3_body,
        grid=(_GRID,),
        in_specs=[
            pl.BlockSpec((T, 256), lambda i: (i, 0)),
            pl.BlockSpec((2048, 256), lambda i: (0, 0)),
            pl.BlockSpec((2048,), lambda i: (0,)),
            pl.BlockSpec((640, 2048), lambda i: (0, 0)),
            pl.BlockSpec((640,), lambda i: (0,)),
        ],
        out_specs=pl.BlockSpec((T, 640), lambda i: (i, 0)),
        out_shape=jax.ShapeDtypeStruct((_N, 640), _f32),
    )(g1, g2w, g2b, gfw, gfb)


def _tc4_body(acc_ref, h_ref, cnt_ref, wl_ref, wr_ref, b_ref, nc_ref):
    rc = _rc_from_cnt(cnt_ref)
    wl = wl_ref[...]
    wr = wr_ref[...]
    nc_ref[...] = (_dot_t(acc_ref[0] * rc, wl[:, :128]) +
                   _dot_t(acc_ref[1] * rc, wl[:, 128:]) +
                   _dot_t(h_ref[0], wr[:, :128]) +
                   _dot_t(h_ref[1], wr[:, 128:]) + b_ref[...][None, :])


def _tc4(acc, h, cnt, wl, wr, b):
    T = _TILE
    row3 = pl.BlockSpec((2, T, 128), lambda i: (0, i, 0))
    return pl.pallas_call(
        _tc4_body,
        grid=(_GRID,),
        in_specs=[
            row3, row3,
            pl.BlockSpec((T, 128), lambda i: (i, 0)),
            pl.BlockSpec((128, 256), lambda i: (0, 0)),
            pl.BlockSpec((128, 256), lambda i: (0, 0)),
            pl.BlockSpec((128,), lambda i: (0,)),
        ],
        out_specs=pl.BlockSpec((T, 128), lambda i: (i, 0)),
        out_shape=jax.ShapeDtypeStruct((_N, 128), _f32),
    )(acc, h, cnt, wl, wr, b)


_B = 128
_EP = 327680
_HP = 5120
_AR = 5248
_RIN = _AR // 16
_RWB = _HP // 16
_K = 4
_NSB = _EP // _B // _K // 16


@functools.lru_cache(maxsize=None)
def _sc_mesh():
    return plsc.VectorSubcoreMesh(core_axis_name="c", subcore_axis_name="s")


def _remap_dst(dstv, part_base):
    basev = jnp.full((16,), part_base, jnp.int32)
    dumpv = jnp.full((16,), _HP, jnp.int32)
    for j in range(_B // 16):
        sl = pl.ds(j * 16, 16)
        local = dstv[sl] - basev
        ok = (local >= 0) & (local < _HP)
        dstv[sl] = jnp.where(ok, local, dumpv)


def _agg_pass(c, s, tab_h, src_h, dst_h, srcv, dstv, rows, accs, semg, sems,
              part_base, src_off):
    def blk(i, carry):
        m = s + 16 * i
        pltpu.sync_copy(src_h.at[pl.ds(m * _K, _K)], srcv)
        if src_off is not None:
            for j in range(_K):
                for q in range(_B // 16):
                    sl = pl.ds(q * 16, 16)
                    srcv[j, sl] = srcv[j, sl] + src_off
        gd = [pltpu.async_copy(tab_h.at[srcv.at[j]], rows.at[j], semg)
              for j in range(_K)]
        pltpu.sync_copy(dst_h.at[pl.ds(m * _K, _K)], dstv)
        for j in range(_K):
            _remap_dst(dstv.at[j], part_base)
        sd = []
        for j in range(_K):
            gd[j].wait()
            sd.append(pltpu.async_copy(rows.at[j], accs.at[dstv.at[j]],
                                       sems, add=True))
        for d in sd:
            d.wait()
        return carry

    lax.fori_loop(0, _NSB, blk, 0)


def _slab_init(s, z_h, dst):
    r0 = s * _RIN
    pltpu.sync_copy(z_h.at[pl.ds(r0, _RIN)], dst.at[pl.ds(r0, _RIN)])


def _slab_wb(s, src, out_h, row_base, lead=None):
    r0 = s * _RWB
    if lead is None:
        dst = out_h.at[pl.ds(row_base + r0, _RWB)]
    else:
        dst = out_h.at[lead, pl.ds(row_base + r0, _RWB)]
    pltpu.sync_copy(src.at[pl.ds(r0, _RWB)], dst)


def _sc0_body(ones_h, dst_h, zacc_h, cnt_o, dstv, onesv, accs, sems):
    c = lax.axis_index("c")
    s = lax.axis_index("s")
    _slab_init(s, zacc_h, accs)
    pltpu.sync_copy(ones_h, onesv)
    plsc.subcore_barrier()

    def blk(i, carry):
        m = s + 16 * i
        pltpu.sync_copy(dst_h.at[pl.ds(m * _K, _K)], dstv)
        for j in range(_K):
            _remap_dst(dstv.at[j], c * _HP)
        sd = [pltpu.async_copy(onesv, accs.at[dstv.at[j]], sems, add=True)
              for j in range(_K)]
        for d in sd:
            d.wait()
        return carry

    lax.fori_loop(0, _NSB, blk, 0)
    plsc.subcore_barrier()
    _slab_wb(s, accs, cnt_o, c * _HP)


def _sc0(ones128, dst, zacc):
    f = functools.partial(
        pl.kernel,
        out_type=jax.ShapeDtypeStruct((2 * _HP, 128), _f32),
        mesh=_sc_mesh(),
        scratch_types=[
            pltpu.VMEM((_K, _B), jnp.int32),
            pltpu.VMEM((_B, 128), _f32),
            pltpu.VMEM_SHARED((_AR, 128), _f32),
            pltpu.SemaphoreType.DMA,
        ],
    )(_sc0_body)
    return f(ones128, dst, zacc)


def _sc1_body(x_h, src_h, dst_h, zacc_h, acc_o, srcv, dstv, rows, accs,
              semg, sems):
    c = lax.axis_index("c")
    s = lax.axis_index("s")
    _slab_init(s, zacc_h, accs)
    plsc.subcore_barrier()
    _agg_pass(c, s, x_h, src_h, dst_h, srcv, dstv, rows, accs, semg, sems,
              part_base=c * _HP, src_off=None)
    plsc.subcore_barrier()
    _slab_wb(s, accs, acc_o, c * _HP)


def _sc1(x, src, dst, zacc):
    f = functools.partial(
        pl.kernel,
        out_type=jax.ShapeDtypeStruct((2 * _HP, 128), _f32),
        mesh=_sc_mesh(),
        scratch_types=[
            pltpu.VMEM((_K, _B), jnp.int32),
            pltpu.VMEM((_K, _B), jnp.int32),
            pltpu.VMEM((_K, _B, 128), _f32),
            pltpu.VMEM_SHARED((_AR, 128), _f32),
            pltpu.SemaphoreType.DMA,
            pltpu.SemaphoreType.DMA,
        ],
    )(_sc1_body)
    return f(x, src, dst, zacc)


def _sc2_body(tab_h, src_h, dst_h, zacc_h, acc_o,
              srcv, dstv, rows, accs, semg, sems):
    c = lax.axis_index("c")
    s = lax.axis_index("s")
    src_off = jnp.full((16,), c * _N, jnp.int32)
    for p in (0, 1):
        _slab_init(s, zacc_h, accs)
        plsc.subcore_barrier()
        _agg_pass(c, s, tab_h, src_h, dst_h, srcv, dstv, rows, accs,
                  semg, sems, part_base=p * _HP, src_off=src_off)
        plsc.subcore_barrier()
        _slab_wb(s, accs, acc_o, p * _HP, lead=c)
        plsc.subcore_barrier()


def _sc2(tab, src, dst, zacc):
    f = functools.partial(
        pl.kernel,
        out_type=jax.ShapeDtypeStruct((2, 2 * _HP, 128), _f32),
        mesh=_sc_mesh(),
        scratch_types=[
            pltpu.VMEM((_K, _B), jnp.int32),
            pltpu.VMEM((_K, _B), jnp.int32),
            pltpu.VMEM((_K, _B, 128), _f32),
            pltpu.VMEM_SHARED((_AR, 128), _f32),
            pltpu.SemaphoreType.DMA,
            pltpu.SemaphoreType.DMA,
        ],
    )(_sc2_body)
    return f(tab, src, dst, zacc)


def kernel(x, edge_index, enc_Wl1, enc_Wr1, enc_b1, enc_Wl2, enc_Wr2, enc_b2,
           reg_W, reg_b, g1_W, g1_b, g2_W, g2_b, gf_W, gf_b,
           cls_Wl1, cls_Wr1, cls_b1, cls_Wl2, cls_Wr2, cls_b2, noise):
    src = edge_index[0].astype(jnp.int32)
    dst = edge_index[1].astype(jnp.int32)
    pad = _EP - _E
    src = jnp.concatenate([src, jnp.zeros((pad,), jnp.int32)])
    dst = jnp.concatenate([dst, jnp.full((pad,), _N, jnp.int32)])
    src = src.reshape(_EP // _B, _B)
    dst = dst.reshape(_EP // _B, _B)
    zacc = jnp.zeros((_AR, 128), _f32)
    ones128 = jnp.ones((_B, 128), _f32)

    cnt = _sc0(ones128, dst, zacc)
    acc1 = _sc1(x, src, dst, zacc)
    h_enc, h_cls = _tc1(acc1, cnt, x, enc_Wl1, enc_Wr1, enc_b1,
                        cls_Wl1, cls_Wr1, cls_b1)
    acc2e = _sc2(h_enc.reshape(2 * _N, 128), src, dst, zacc)
    acc2c = _sc2(h_cls.reshape(2 * _N, 128), src, dst, zacc)
    degree, g1 = _tc2(acc2e, h_enc, cnt, noise, enc_Wl2, enc_Wr2, enc_b2,
                      reg_W, reg_b, g1_W, g1_b)
    nc = _tc4(acc2c, h_cls, cnt, cls_Wl2, cls_Wr2, cls_b2)
    gen_feat = _tc3(g1, g2_W, g2_b, gf_W, gf_b)
    return degree, gen_feat, nc


# speedup vs baseline: 1.0767x; 1.0271x over previous
import functools

import jax
import jax.numpy as jnp
from jax import lax
from jax.experimental import pallas as pl
from jax.experimental.pallas import tpu as pltpu
from jax.experimental.pallas import tpu_sc as plsc

_N = 10000
_E = 320000
_TILE = 400
_GRID = _N // _TILE

_f32 = jnp.float32


def _dot_t(a, w):
    return lax.dot_general(a, w, (((1,), (1,)), ((), ())),
                           preferred_element_type=_f32)


def _rc_from_cnt(cnt_ref):
    cnt = cnt_ref[...][:, 0:1]
    return 1.0 / jnp.maximum(cnt, 1.0)


def _tc1_body(acc_ref, cnt_ref, x_ref, ewl_ref, ewr_ref, eb_ref,
              cwl_ref, cwr_ref, cb_ref, he_ref, hc_ref):
    rc = _rc_from_cnt(cnt_ref)
    agg = acc_ref[...] * rc
    x = x_ref[...]
    he = jnp.maximum(_dot_t(agg, ewl_ref[...]) + _dot_t(x, ewr_ref[...])
                     + eb_ref[...][None, :], 0.0)
    hc = jnp.maximum(_dot_t(agg, cwl_ref[...]) + _dot_t(x, cwr_ref[...])
                     + cb_ref[...][None, :], 0.0)
    he_ref[0] = he[:, :128]
    he_ref[1] = he[:, 128:]
    hc_ref[0] = hc[:, :128]
    hc_ref[1] = hc[:, 128:]


def _tc1(acc, cnt, x, ewl, ewr, eb, cwl, cwr, cb):
    T = _TILE
    row3 = pl.BlockSpec((2, T, 128), lambda i: (0, i, 0))
    return pl.pallas_call(
        _tc1_body,
        grid=(_GRID,),
        in_specs=[
            pl.BlockSpec((T, 128), lambda i: (i, 0)),
            pl.BlockSpec((T, 128), lambda i: (i, 0)),
            pl.BlockSpec((T, 128), lambda i: (i, 0)),
            pl.BlockSpec((256, 128), lambda i: (0, 0)),
            pl.BlockSpec((256, 128), lambda i: (0, 0)),
            pl.BlockSpec((256,), lambda i: (0,)),
            pl.BlockSpec((256, 128), lambda i: (0, 0)),
            pl.BlockSpec((256, 128), lambda i: (0, 0)),
            pl.BlockSpec((256,), lambda i: (0,)),
        ],
        out_specs=[row3, row3],
        out_shape=[jax.ShapeDtypeStruct((2, _N, 128), _f32),
                   jax.ShapeDtypeStruct((2, _N, 128), _f32)],
    )(acc, cnt, x, ewl, ewr, eb, cwl, cwr, cb)


def _tc2_body(acc_ref, h_ref, cnt_ref, noise_ref, wl_ref, wr_ref, b_ref,
              regw_ref, regb_ref, g1w_ref, g1b_ref, deg_ref, g1_ref):
    rc = _rc_from_cnt(cnt_ref)
    wl = wl_ref[...]
    wr = wr_ref[...]
    z = (_dot_t(acc_ref[0] * rc, wl[:, :128]) +
         _dot_t(acc_ref[1] * rc, wl[:, 128:]) +
         _dot_t(h_ref[0], wr[:, :128]) +
         _dot_t(h_ref[1], wr[:, 128:]) + b_ref[...][None, :])
    d = jnp.sum(z * regw_ref[...], axis=1, keepdims=True) + regb_ref[0]
    deg_ref[...] = jnp.maximum(d, 0.0)
    g1_ref[...] = jnp.maximum(_dot_t(z + noise_ref[...], g1w_ref[...])
                              + g1b_ref[...][None, :], 0.0)


def _tc2(acc, h, cnt, noise, wl, wr, b, regw, regb, g1w, g1b):
    T = _TILE
    row3 = pl.BlockSpec((2, T, 128), lambda i: (0, i, 0))
    return pl.pallas_call(
        _tc2_body,
        grid=(_GRID,),
        in_specs=[
            row3, row3,
            pl.BlockSpec((T, 128), lambda i: (i, 0)),
            pl.BlockSpec((T, 128), lambda i: (i, 0)),
            pl.BlockSpec((128, 256), lambda i: (0, 0)),
            pl.BlockSpec((128, 256), lambda i: (0, 0)),
            pl.BlockSpec((128,), lambda i: (0,)),
            pl.BlockSpec((1, 128), lambda i: (0, 0)),
            pl.BlockSpec((1,), lambda i: (0,)),
            pl.BlockSpec((256, 128), lambda i: (0, 0)),
            pl.BlockSpec((256,), lambda i: (0,)),
        ],
        out_specs=[pl.BlockSpec((T, 1), lambda i: (i, 0)),
                   pl.BlockSpec((T, 256), lambda i: (i, 0))],
        out_shape=[jax.ShapeDtypeStruct((_N, 1), _f32),
                   jax.ShapeDtypeStruct((_N, 256), _f32)],
    )(acc, h, cnt, noise, wl, wr, b, regw, regb, g1w, g1b)


def _tc3_body(g1_ref, g2w_ref, g2b_ref, gfw_ref, gfb_ref, out_ref):
    g2 = jnp.maximum(_dot_t(g1_ref[...], g2w_ref[...]) + g2b_ref[...][None, :],
                     0.0)
    out_ref[...] = jnp.tanh(_dot_t(g2, gfw_ref[...]) + gfb_ref[...][None, :])


def _tc3(g1, g2w, g2b, gfw, gfb):
    T = _TILE
    return pl.pallas_call(
        _tc3_body,
        grid=(_GRID,),
        in_specs=[
            pl.BlockSpec((T, 256), lambda i: (i, 0)),
            pl.BlockSpec((2048, 256), lambda i: (0, 0)),
            pl.BlockSpec((2048,), lambda i: (0,)),
            pl.BlockSpec((640, 2048), lambda i: (0, 0)),
            pl.BlockSpec((640,), lambda i: (0,)),
        ],
        out_specs=pl.BlockSpec((T, 640), lambda i: (i, 0)),
        out_shape=jax.ShapeDtypeStruct((_N, 640), _f32),
    )(g1, g2w, g2b, gfw, gfb)


def _tc4_body(acc_ref, h_ref, cnt_ref, wl_ref, wr_ref, b_ref, nc_ref):
    rc = _rc_from_cnt(cnt_ref)
    wl = wl_ref[...]
    wr = wr_ref[...]
    nc_ref[...] = (_dot_t(acc_ref[0] * rc, wl[:, :128]) +
                   _dot_t(acc_ref[1] * rc, wl[:, 128:]) +
                   _dot_t(h_ref[0], wr[:, :128]) +
                   _dot_t(h_ref[1], wr[:, 128:]) + b_ref[...][None, :])


def _tc4(acc, h, cnt, wl, wr, b):
    T = _TILE
    row3 = pl.BlockSpec((2, T, 128), lambda i: (0, i, 0))
    return pl.pallas_call(
        _tc4_body,
        grid=(_GRID,),
        in_specs=[
            row3, row3,
            pl.BlockSpec((T, 128), lambda i: (i, 0)),
            pl.BlockSpec((128, 256), lambda i: (0, 0)),
            pl.BlockSpec((128, 256), lambda i: (0, 0)),
            pl.BlockSpec((128,), lambda i: (0,)),
        ],
        out_specs=pl.BlockSpec((T, 128), lambda i: (i, 0)),
        out_shape=jax.ShapeDtypeStruct((_N, 128), _f32),
    )(acc, h, cnt, wl, wr, b)


_B = 128
_EP = 323584
_HP = 5120
_AR = 5248
_RIN = _AR // 16
_RWB = _HP // 16
_NBLK = _EP // _B // 16


@functools.lru_cache(maxsize=None)
def _sc_mesh():
    return plsc.VectorSubcoreMesh(core_axis_name="c", subcore_axis_name="s")


def _remap_dst(dstv, part_base):
    basev = jnp.full((16,), part_base, jnp.int32)
    dumpv = jnp.full((16,), _HP, jnp.int32)
    for j in range(_B // 16):
        sl = pl.ds(j * 16, 16)
        local = dstv[sl] - basev
        ok = (local >= 0) & (local < _HP)
        dstv[sl] = jnp.where(ok, local, dumpv)


def _agg_pass(c, s, tab_h, src_h, dst_h, srcv, dstv, rows, accs, sem,
              part_base, src_off):
    def blk(i, carry):
        off = (s + 16 * i) * _B
        pltpu.sync_copy(src_h.at[pl.ds(off, _B)], srcv)
        if src_off is not None:
            for j in range(_B // 16):
                sl = pl.ds(j * 16, 16)
                srcv[sl] = srcv[sl] + src_off
        pltpu.async_copy(tab_h.at[srcv], rows, sem).wait()
        pltpu.sync_copy(dst_h.at[pl.ds(off, _B)], dstv)
        _remap_dst(dstv, part_base)
        pltpu.sync_copy(rows, accs.at[dstv], add=True)
        return carry

    lax.fori_loop(0, _NBLK, blk, 0)


def _slab_init(s, z_h, dst):
    r0 = s * _RIN
    pltpu.sync_copy(z_h.at[pl.ds(r0, _RIN)], dst.at[pl.ds(r0, _RIN)])


def _slab_wb(s, src, out_h, row_base, lead=None):
    r0 = s * _RWB
    if lead is None:
        dst = out_h.at[pl.ds(row_base + r0, _RWB)]
    else:
        dst = out_h.at[lead, pl.ds(row_base + r0, _RWB)]
    pltpu.sync_copy(src.at[pl.ds(r0, _RWB)], dst)


def _sc0_body(ones_h, dst_h, zacc_h, cnt_o, dstv, onesv, accs):
    c = lax.axis_index("c")
    s = lax.axis_index("s")
    _slab_init(s, zacc_h, accs)
    pltpu.sync_copy(ones_h, onesv)
    plsc.subcore_barrier()

    def blk(i, carry):
        off = (s + 16 * i) * _B
        pltpu.sync_copy(dst_h.at[pl.ds(off, _B)], dstv)
        _remap_dst(dstv, c * _HP)
        pltpu.sync_copy(onesv, accs.at[dstv], add=True)
        return carry

    lax.fori_loop(0, _NBLK, blk, 0)
    plsc.subcore_barrier()
    _slab_wb(s, accs, cnt_o, c * _HP)


def _sc0(ones128, dst, zacc):
    f = functools.partial(
        pl.kernel,
        out_type=jax.ShapeDtypeStruct((2 * _HP, 128), _f32),
        mesh=_sc_mesh(),
        scratch_types=[
            pltpu.VMEM((_B,), jnp.int32),
            pltpu.VMEM((_B, 128), _f32),
            pltpu.VMEM_SHARED((_AR, 128), _f32),
        ],
    )(_sc0_body)
    return f(ones128, dst, zacc)


def _sc1_body(x_h, src_h, dst_h, zacc_h, acc_o, srcv, dstv, rows, accs, sem):
    c = lax.axis_index("c")
    s = lax.axis_index("s")
    _slab_init(s, zacc_h, accs)
    plsc.subcore_barrier()
    _agg_pass(c, s, x_h, src_h, dst_h, srcv, dstv, rows, accs, sem,
              part_base=c * _HP, src_off=None)
    plsc.subcore_barrier()
    _slab_wb(s, accs, acc_o, c * _HP)


def _sc1(x, src, dst, zacc):
    f = functools.partial(
        pl.kernel,
        out_type=jax.ShapeDtypeStruct((2 * _HP, 128), _f32),
        mesh=_sc_mesh(),
        scratch_types=[
            pltpu.VMEM((_B,), jnp.int32),
            pltpu.VMEM((_B,), jnp.int32),
            pltpu.VMEM((_B, 128), _f32),
            pltpu.VMEM_SHARED((_AR, 128), _f32),
            pltpu.SemaphoreType.DMA,
        ],
    )(_sc1_body)
    return f(x, src, dst, zacc)


def _sc2_body(tab_h, src_h, dst_h, zacc_h, acc_o,
              srcv, dstv, rows, accs, sem):
    c = lax.axis_index("c")
    s = lax.axis_index("s")
    src_off = jnp.full((16,), c * _N, jnp.int32)
    for p in (0, 1):
        _slab_init(s, zacc_h, accs)
        plsc.subcore_barrier()
        _agg_pass(c, s, tab_h, src_h, dst_h, srcv, dstv, rows, accs, sem,
                  part_base=p * _HP, src_off=src_off)
        plsc.subcore_barrier()
        _slab_wb(s, accs, acc_o, p * _HP, lead=c)
        plsc.subcore_barrier()


def _sc2(tab, src, dst, zacc):
    f = functools.partial(
        pl.kernel,
        out_type=jax.ShapeDtypeStruct((2, 2 * _HP, 128), _f32),
        mesh=_sc_mesh(),
        scratch_types=[
            pltpu.VMEM((_B,), jnp.int32),
            pltpu.VMEM((_B,), jnp.int32),
            pltpu.VMEM((_B, 128), _f32),
            pltpu.VMEM_SHARED((_AR, 128), _f32),
            pltpu.SemaphoreType.DMA,
        ],
    )(_sc2_body)
    return f(tab, src, dst, zacc)


def kernel(x, edge_index, enc_Wl1, enc_Wr1, enc_b1, enc_Wl2, enc_Wr2, enc_b2,
           reg_W, reg_b, g1_W, g1_b, g2_W, g2_b, gf_W, gf_b,
           cls_Wl1, cls_Wr1, cls_b1, cls_Wl2, cls_Wr2, cls_b2, noise):
    src = edge_index[0].astype(jnp.int32)
    dst = edge_index[1].astype(jnp.int32)
    pad = _EP - _E
    src = jnp.concatenate([src, jnp.zeros((pad,), jnp.int32)])
    dst = jnp.concatenate([dst, jnp.full((pad,), _N, jnp.int32)])
    zacc = jnp.zeros((_AR, 128), _f32)
    ones128 = jnp.ones((_B, 128), _f32)

    cnt = _sc0(ones128, dst, zacc)
    acc1 = _sc1(x, src, dst, zacc)
    h_enc, h_cls = _tc1(acc1, cnt, x, enc_Wl1, enc_Wr1, enc_b1,
                        cls_Wl1, cls_Wr1, cls_b1)
    acc2e = _sc2(h_enc.reshape(2 * _N, 128), src, dst, zacc)
    acc2c = _sc2(h_cls.reshape(2 * _N, 128), src, dst, zacc)
    degree, g1 = _tc2(acc2e, h_enc, cnt, noise, enc_Wl2, enc_Wr2, enc_b2,
                      reg_W, reg_b, g1_W, g1_b)
    nc = _tc4(acc2c, h_cls, cnt, cls_Wl2, cls_Wr2, cls_b2)
    gen_feat = _tc3(g1, g2_W, g2_b, gf_W, gf_b)
    return degree, gen_feat, nc
